# baseline (speedup 1.0000x reference)
import functools

import jax
import jax.numpy as jnp
from jax import lax
from jax.experimental import pallas as pl
from jax.experimental.pallas import tpu as pltpu
from jax.experimental.pallas import tpu_sc as plsc

_NC = 2
_NS = 16
_NW = _NC * _NS


@functools.partial(jax.jit, static_argnames=("b", "c", "t", "h", "w"))
def _avg_pool(x, b, c, t, h, w):
    xt = lax.transpose(x, (0, 2, 3, 4, 1))
    n = b * c * t
    pairs = b * t
    ppw = pairs // _NW
    nch = 2 * ppw
    cg = c // 16
    n_valid = (h - 1) * (h - 2)
    inv = 1.0 / float(n_valid)

    mesh = plsc.VectorSubcoreMesh(core_axis_name="c", subcore_axis_name="s")

    @functools.partial(
        pl.kernel,
        out_type=jax.ShapeDtypeStruct((n,), jnp.float32),
        mesh=mesh,
        scratch_types=[
            pltpu.VMEM((5, h, c), jnp.float32),
            pltpu.VMEM((5, h, c), jnp.float32),
            pltpu.VMEM((5, h, c), jnp.float32),
            pltpu.VMEM((5, h, c), jnp.float32),
            pltpu.VMEM((ppw * c,), jnp.float32),
            pltpu.SemaphoreType.DMA,
            pltpu.SemaphoreType.DMA,
            pltpu.SemaphoreType.DMA,
            pltpu.SemaphoreType.DMA,
        ],
    )
    def sc_kernel(xt_hbm, out_hbm, b0, b1, b2, b3, outbuf, s0, s1, s2, s3):
        wid = lax.axis_index("s") * _NC + lax.axis_index("c")
        p0 = wid * ppw
        bufs = [b0, b1, b2, b3]
        sems = [s0, s1, s2, s3]
        def src(pi, cj):
            p = p0 + pi
            return xt_hbm.at[p // t, p % t, pl.ds(5 * cj, 5)]

        def start(pi, cj, sl):
            @pl.when(pi < ppw)
            def _():
                pltpu.async_copy(src(pi, cj), bufs[sl], sems[sl])

        def wait(pi, cj, sl):
            pltpu.make_async_copy(src(pi, cj), bufs[sl], sems[sl]).wait()

        def compute(pi, cj, sl):
            buf = bufs[sl]

            @pl.loop(0, cg)
            def _cgrp(k):
                o = pl.ds(pi * c + k * 16, 16)
                vs = [
                    buf[hh, ww, pl.ds(k * 16, 16)]
                    for hh in range(5)
                    for ww in range(1, w - 1)
                ]
                if cj:
                    vs.append(outbuf[o])
                while len(vs) > 1:
                    nxt = [
                        vs[i] + vs[i + 1] for i in range(0, len(vs) - 1, 2)
                    ]
                    if len(vs) % 2:
                        nxt.append(vs[-1])
                    vs = nxt
                outbuf[o] = vs[0] * inv if cj == 2 else vs[0]

        for g in range(3):
            start(g // 3, g % 3, g % 4)

        @pl.loop(0, ppw, step=4)
        def _plane4(pb):
            for q in range(12):
                nq = q + 3
                start(pb + nq // 3, nq % 3, nq % 4)
                wait(pb + q // 3, q % 3, q % 4)
                compute(pb + q // 3, q % 3, q % 4)

        pltpu.sync_copy(outbuf, out_hbm.at[pl.ds(p0 * c, ppw * c)])

    out = sc_kernel(xt)
    return lax.transpose(out.reshape(b, t, c), (0, 2, 1))


def kernel(x):
    b, c, t, h, w = x.shape
    assert h == 16 and w == 16, "kernel specialized to 16x16 spatial grids"
    assert (b * t) % _NW == 0 and c % 16 == 0
    return _avg_pool(x, b, c, t, h, w)


# speedup vs baseline: 1.1505x; 1.1505x over previous
import functools

import jax
import jax.numpy as jnp
from jax import lax
from jax.experimental import pallas as pl
from jax.experimental.pallas import tpu as pltpu
from jax.experimental.pallas import tpu_sc as plsc

_NC = 2
_NS = 16
_NW = _NC * _NS


_SC_FRAC_NUM, _SC_FRAC_DEN = 5, 8


@functools.partial(jax.jit, static_argnames=("b", "c", "t", "h", "w"))
def _avg_pool(x, b, c, t, h, w):
    xt = lax.transpose(x, (0, 2, 3, 4, 1))
    pairs = b * t
    p_sc = pairs * _SC_FRAC_NUM // _SC_FRAC_DEN // _NW * _NW
    p_tc = pairs - p_sc
    n = p_sc * c
    ppw = p_sc // _NW
    nch = 2 * ppw
    cg = c // 16
    n_valid = (h - 1) * (h - 2)
    inv = 1.0 / float(n_valid)

    mesh = plsc.VectorSubcoreMesh(core_axis_name="c", subcore_axis_name="s")

    @functools.partial(
        pl.kernel,
        out_type=jax.ShapeDtypeStruct((n,), jnp.float32),
        mesh=mesh,
        scratch_types=[
            pltpu.VMEM((4, h, c), jnp.float32),
            pltpu.VMEM((4, h, c), jnp.float32),
            pltpu.VMEM((4, h, c), jnp.float32),
            pltpu.VMEM((3, h, c), jnp.float32),
            pltpu.VMEM((ppw * c,), jnp.float32),
            pltpu.SemaphoreType.DMA,
            pltpu.SemaphoreType.DMA,
            pltpu.SemaphoreType.DMA,
            pltpu.SemaphoreType.DMA,
        ],
    )
    def sc_kernel(xt_hbm, out_hbm, b0, b1, b2, b3, outbuf, s0, s1, s2, s3):
        wid = lax.axis_index("s") * _NC + lax.axis_index("c")
        p0 = wid * ppw
        bufs = [b0, b1, b2, b3]
        sems = [s0, s1, s2, s3]
        h0s = (0, 4, 8, 12)
        hss = (range(4), range(4), range(4), range(3))
        hns = (4, 4, 4, 3)

        def src(pi, cj):
            p = p0 + pi
            return xt_hbm.at[p // t, p % t, pl.ds(h0s[cj], hns[cj])]

        def start(pi, cj):
            @pl.when(pi < ppw)
            def _():
                pltpu.async_copy(src(pi, cj), bufs[cj], sems[cj])

        def wait(pi, cj):
            pltpu.make_async_copy(src(pi, cj), bufs[cj], sems[cj]).wait()

        def compute(pi, cj):
            buf = bufs[cj]

            @pl.loop(0, cg)
            def _cgrp(k):
                o = pl.ds(pi * c + k * 16, 16)
                vs = [
                    buf[hh, ww, pl.ds(k * 16, 16)]
                    for hh in hss[cj]
                    for ww in range(1, w - 1)
                ]
                if cj:
                    vs.append(outbuf[o])
                while len(vs) > 1:
                    nxt = [
                        vs[i] + vs[i + 1] for i in range(0, len(vs) - 1, 2)
                    ]
                    if len(vs) % 2:
                        nxt.append(vs[-1])
                    vs = nxt
                outbuf[o] = vs[0] * inv if cj == 3 else vs[0]

        for cj in range(3):
            start(0, cj)

        @pl.loop(0, ppw)
        def _plane(pi):
            for cj in range(4):
                nxt = cj + 3
                start(pi + nxt // 4, nxt % 4)
                wait(pi, cj)
                compute(pi, cj)

        pltpu.sync_copy(outbuf, out_hbm.at[pl.ds(p0 * c, ppw * c)])

    xt5 = xt.reshape(pairs, h, w, c)

    def tc_body(x_ref, o_ref):
        wm = jnp.logical_and(
            lax.broadcasted_iota(jnp.int32, (1, 1, w, 1), 2) >= 1,
            lax.broadcasted_iota(jnp.int32, (1, 1, w, 1), 2) <= w - 2,
        )
        xm = jnp.where(wm, x_ref[:, : h - 1], 0.0)
        o_ref[...] = jnp.sum(xm, axis=(1, 2)) * inv

    tcb = 8
    out_tc = pl.pallas_call(
        tc_body,
        grid=(p_tc // tcb,),
        in_specs=[
            pl.BlockSpec((tcb, h, w, c), lambda p: (p_sc // tcb + p, 0, 0, 0)),
        ],
        out_specs=pl.BlockSpec((tcb, c), lambda p: (p, 0)),
        out_shape=jax.ShapeDtypeStruct((p_tc, c), jnp.float32),
    )(xt5)

    out_sc = sc_kernel(xt)
    out = jnp.concatenate([out_sc.reshape(p_sc, c), out_tc], axis=0)
    return lax.transpose(out.reshape(b, t, c), (0, 2, 1))


def kernel(x):
    b, c, t, h, w = x.shape
    assert h == 16 and w == 16, "kernel specialized to 16x16 spatial grids"
    assert (b * t) % _NW == 0 and c % 16 == 0
    return _avg_pool(x, b, c, t, h, w)


# speedup vs baseline: 1.1510x; 1.0004x over previous
import functools

import jax
import jax.numpy as jnp
from jax import lax
from jax.experimental import pallas as pl
from jax.experimental.pallas import tpu as pltpu
from jax.experimental.pallas import tpu_sc as plsc

_NC = 2
_NS = 16
_NW = _NC * _NS


_SC_FRAC_NUM, _SC_FRAC_DEN = 4, 8


@functools.partial(jax.jit, static_argnames=("b", "c", "t", "h", "w"))
def _avg_pool(x, b, c, t, h, w):
    xt = lax.transpose(x, (0, 2, 3, 4, 1))
    pairs = b * t
    p_sc = pairs * _SC_FRAC_NUM // _SC_FRAC_DEN // _NW * _NW
    p_tc = pairs - p_sc
    n = p_sc * c
    ppw = p_sc // _NW
    nch = 2 * ppw
    cg = c // 16
    n_valid = (h - 1) * (h - 2)
    inv = 1.0 / float(n_valid)

    mesh = plsc.VectorSubcoreMesh(core_axis_name="c", subcore_axis_name="s")

    @functools.partial(
        pl.kernel,
        out_type=jax.ShapeDtypeStruct((n,), jnp.float32),
        mesh=mesh,
        scratch_types=[
            pltpu.VMEM((4, h, c), jnp.float32),
            pltpu.VMEM((4, h, c), jnp.float32),
            pltpu.VMEM((4, h, c), jnp.float32),
            pltpu.VMEM((3, h, c), jnp.float32),
            pltpu.VMEM((ppw * c,), jnp.float32),
            pltpu.SemaphoreType.DMA,
            pltpu.SemaphoreType.DMA,
            pltpu.SemaphoreType.DMA,
            pltpu.SemaphoreType.DMA,
        ],
    )
    def sc_kernel(xt_hbm, out_hbm, b0, b1, b2, b3, outbuf, s0, s1, s2, s3):
        wid = lax.axis_index("s") * _NC + lax.axis_index("c")
        p0 = wid * ppw
        bufs = [b0, b1, b2, b3]
        sems = [s0, s1, s2, s3]
        h0s = (0, 4, 8, 12)
        hss = (range(4), range(4), range(4), range(3))
        hns = (4, 4, 4, 3)

        def src(pi, cj):
            p = p0 + pi
            return xt_hbm.at[p // t, p % t, pl.ds(h0s[cj], hns[cj])]

        def start(pi, cj):
            @pl.when(pi < ppw)
            def _():
                pltpu.async_copy(src(pi, cj), bufs[cj], sems[cj])

        def wait(pi, cj):
            pltpu.make_async_copy(src(pi, cj), bufs[cj], sems[cj]).wait()

        def compute(pi, cj):
            buf = bufs[cj]

            @pl.loop(0, cg)
            def _cgrp(k):
                o = pl.ds(pi * c + k * 16, 16)
                vs = [
                    buf[hh, ww, pl.ds(k * 16, 16)]
                    for hh in hss[cj]
                    for ww in range(1, w - 1)
                ]
                if cj:
                    vs.append(outbuf[o])
                while len(vs) > 1:
                    nxt = [
                        vs[i] + vs[i + 1] for i in range(0, len(vs) - 1, 2)
                    ]
                    if len(vs) % 2:
                        nxt.append(vs[-1])
                    vs = nxt
                outbuf[o] = vs[0] * inv if cj == 3 else vs[0]

        for cj in range(3):
            start(0, cj)

        @pl.loop(0, ppw)
        def _plane(pi):
            for cj in range(4):
                nxt = cj + 3
                start(pi + nxt // 4, nxt % 4)
                wait(pi, cj)
                compute(pi, cj)

        pltpu.sync_copy(outbuf, out_hbm.at[pl.ds(p0 * c, ppw * c)])

    xt5 = xt.reshape(pairs, h, w, c)

    def tc_body(x_ref, o_ref):
        wm = jnp.logical_and(
            lax.broadcasted_iota(jnp.int32, (1, 1, w, 1), 2) >= 1,
            lax.broadcasted_iota(jnp.int32, (1, 1, w, 1), 2) <= w - 2,
        )
        xm = jnp.where(wm, x_ref[:, : h - 1], 0.0)
        o_ref[...] = jnp.sum(xm, axis=(1, 2)) * inv

    tcb = 8
    out_tc = pl.pallas_call(
        tc_body,
        grid=(p_tc // tcb,),
        in_specs=[
            pl.BlockSpec((tcb, h, w, c), lambda p: (p_sc // tcb + p, 0, 0, 0)),
        ],
        out_specs=pl.BlockSpec((tcb, c), lambda p: (p, 0)),
        out_shape=jax.ShapeDtypeStruct((p_tc, c), jnp.float32),
    )(xt5)

    out_sc = sc_kernel(xt)
    out = jnp.concatenate([out_sc.reshape(p_sc, c), out_tc], axis=0)
    return lax.transpose(out.reshape(b, t, c), (0, 2, 1))


def kernel(x):
    b, c, t, h, w = x.shape
    assert h == 16 and w == 16, "kernel specialized to 16x16 spatial grids"
    assert (b * t) % _NW == 0 and c % 16 == 0
    return _avg_pool(x, b, c, t, h, w)


# speedup vs baseline: 1.1714x; 1.0177x over previous
import functools

import jax
import jax.numpy as jnp
from jax import lax
from jax.experimental import pallas as pl
from jax.experimental.pallas import tpu as pltpu
from jax.experimental.pallas import tpu_sc as plsc

_NC = 2
_NS = 16
_NW = _NC * _NS


_SC_FRAC_NUM, _SC_FRAC_DEN = 4, 8


@functools.partial(jax.jit, static_argnames=("b", "c", "t", "h", "w"))
def _avg_pool(x, b, c, t, h, w):
    xt = lax.transpose(x, (0, 2, 3, 4, 1))
    pairs = b * t
    p_sc = pairs * _SC_FRAC_NUM // _SC_FRAC_DEN // _NW * _NW
    p_tc = pairs - p_sc
    n = p_sc * c
    ppw = p_sc // _NW
    nch = 2 * ppw
    cg = c // 16
    n_valid = (h - 1) * (h - 2)
    inv = 1.0 / float(n_valid)

    mesh = plsc.VectorSubcoreMesh(core_axis_name="c", subcore_axis_name="s")

    @functools.partial(
        pl.kernel,
        out_type=jax.ShapeDtypeStruct((n,), jnp.float32),
        mesh=mesh,
        scratch_types=[
            pltpu.VMEM((4, h, c), jnp.float32),
            pltpu.VMEM((4, h, c), jnp.float32),
            pltpu.VMEM((4, h, c), jnp.float32),
            pltpu.VMEM((3, h, c), jnp.float32),
            pltpu.VMEM((ppw * c,), jnp.float32),
            pltpu.SemaphoreType.DMA,
            pltpu.SemaphoreType.DMA,
            pltpu.SemaphoreType.DMA,
            pltpu.SemaphoreType.DMA,
        ],
    )
    def sc_kernel(xt_hbm, out_hbm, b0, b1, b2, b3, outbuf, s0, s1, s2, s3):
        wid = lax.axis_index("s") * _NC + lax.axis_index("c")
        p0 = wid * ppw
        bufs = [b0, b1, b2, b3]
        sems = [s0, s1, s2, s3]
        h0s = (0, 4, 8, 12)
        hss = (range(4), range(4), range(4), range(3))
        hns = (4, 4, 4, 3)

        def src(pi, cj):
            p = p0 + pi
            return xt_hbm.at[p // t, p % t, pl.ds(h0s[cj], hns[cj])]

        def start(pi, cj):
            @pl.when(pi < ppw)
            def _():
                pltpu.async_copy(src(pi, cj), bufs[cj], sems[cj])

        def wait(pi, cj):
            pltpu.make_async_copy(src(pi, cj), bufs[cj], sems[cj]).wait()

        def compute(pi, cj):
            buf = bufs[cj]

            @pl.loop(0, cg)
            def _cgrp(k):
                o = pl.ds(pi * c + k * 16, 16)
                vs = [
                    buf[hh, ww, pl.ds(k * 16, 16)]
                    for hh in hss[cj]
                    for ww in range(1, w - 1)
                ]
                if cj:
                    vs.append(outbuf[o])
                while len(vs) > 1:
                    nxt = [
                        vs[i] + vs[i + 1] for i in range(0, len(vs) - 1, 2)
                    ]
                    if len(vs) % 2:
                        nxt.append(vs[-1])
                    vs = nxt
                outbuf[o] = vs[0] * inv if cj == 3 else vs[0]

        for cj in range(3):
            start(0, cj)

        @pl.loop(0, ppw)
        def _plane(pi):
            for cj in range(4):
                nxt = cj + 3
                start(pi + nxt // 4, nxt % 4)
                wait(pi, cj)
                compute(pi, cj)

        pltpu.sync_copy(outbuf, out_hbm.at[pl.ds(p0 * c, ppw * c)])

    xt5 = xt.reshape(pairs, h, w, c)

    def tc_body(x_ref, o_ref):
        wm = jnp.logical_and(
            lax.broadcasted_iota(jnp.int32, (1, 1, w, 1), 2) >= 1,
            lax.broadcasted_iota(jnp.int32, (1, 1, w, 1), 2) <= w - 2,
        )
        xm = jnp.where(wm, x_ref[:, : h - 1], 0.0)
        o_ref[...] = jnp.sum(xm, axis=(1, 2)) * inv

    tcb = 16
    out_tc = pl.pallas_call(
        tc_body,
        grid=(p_tc // tcb,),
        in_specs=[
            pl.BlockSpec((tcb, h, w, c), lambda p: (p_sc // tcb + p, 0, 0, 0)),
        ],
        out_specs=pl.BlockSpec((tcb, c), lambda p: (p, 0)),
        out_shape=jax.ShapeDtypeStruct((p_tc, c), jnp.float32),
    )(xt5)

    out_sc = sc_kernel(xt)
    out = jnp.concatenate([out_sc.reshape(p_sc, c), out_tc], axis=0)
    return lax.transpose(out.reshape(b, t, c), (0, 2, 1))


def kernel(x):
    b, c, t, h, w = x.shape
    assert h == 16 and w == 16, "kernel specialized to 16x16 spatial grids"
    assert (b * t) % _NW == 0 and c % 16 == 0
    return _avg_pool(x, b, c, t, h, w)


# speedup vs baseline: 1.1758x; 1.0038x over previous
import functools

import jax
import jax.numpy as jnp
from jax import lax
from jax.experimental import pallas as pl
from jax.experimental.pallas import tpu as pltpu
from jax.experimental.pallas import tpu_sc as plsc

_NC = 2
_NS = 16
_NW = _NC * _NS


_SC_FRAC_NUM, _SC_FRAC_DEN = 4, 8


@functools.partial(jax.jit, static_argnames=("b", "c", "t", "h", "w"))
def _avg_pool(x, b, c, t, h, w):
    xt = lax.transpose(x, (0, 2, 3, 4, 1))
    pairs = b * t
    p_sc = pairs * _SC_FRAC_NUM // _SC_FRAC_DEN // _NW * _NW
    p_tc = pairs - p_sc
    n = p_sc * c
    ppw = p_sc // _NW
    nch = 2 * ppw
    cg = c // 16
    n_valid = (h - 1) * (h - 2)
    inv = 1.0 / float(n_valid)

    mesh = plsc.VectorSubcoreMesh(core_axis_name="c", subcore_axis_name="s")

    @functools.partial(
        pl.kernel,
        out_type=jax.ShapeDtypeStruct((n,), jnp.float32),
        mesh=mesh,
        scratch_types=[
            pltpu.VMEM((4, h, c), jnp.float32),
            pltpu.VMEM((4, h, c), jnp.float32),
            pltpu.VMEM((4, h, c), jnp.float32),
            pltpu.VMEM((3, h, c), jnp.float32),
            pltpu.VMEM((ppw * c,), jnp.float32),
            pltpu.SemaphoreType.DMA,
            pltpu.SemaphoreType.DMA,
            pltpu.SemaphoreType.DMA,
            pltpu.SemaphoreType.DMA,
        ],
    )
    def sc_kernel(xt_hbm, out_hbm, b0, b1, b2, b3, outbuf, s0, s1, s2, s3):
        wid = lax.axis_index("s") * _NC + lax.axis_index("c")
        p0 = wid * ppw
        bufs = [b0, b1, b2, b3]
        sems = [s0, s1, s2, s3]
        h0s = (0, 4, 8, 12)
        hss = (range(4), range(4), range(4), range(3))
        hns = (4, 4, 4, 3)

        def src(pi, cj):
            p = p0 + pi
            return xt_hbm.at[p // t, p % t, pl.ds(h0s[cj], hns[cj])]

        def start(pi, cj):
            @pl.when(pi < ppw)
            def _():
                pltpu.async_copy(src(pi, cj), bufs[cj], sems[cj])

        def wait(pi, cj):
            pltpu.make_async_copy(src(pi, cj), bufs[cj], sems[cj]).wait()

        def compute(pi, cj):
            buf = bufs[cj]

            @pl.loop(0, cg)
            def _cgrp(k):
                o = pl.ds(pi * c + k * 16, 16)
                vs = [
                    buf[hh, ww, pl.ds(k * 16, 16)]
                    for hh in hss[cj]
                    for ww in range(1, w - 1)
                ]
                if cj:
                    vs.append(outbuf[o])
                while len(vs) > 1:
                    nxt = [
                        vs[i] + vs[i + 1] for i in range(0, len(vs) - 1, 2)
                    ]
                    if len(vs) % 2:
                        nxt.append(vs[-1])
                    vs = nxt
                outbuf[o] = vs[0] * inv if cj == 3 else vs[0]

        for cj in range(3):
            start(0, cj)

        @pl.loop(0, ppw)
        def _plane(pi):
            for cj in range(4):
                nxt = cj + 3
                start(pi + nxt // 4, nxt % 4)
                wait(pi, cj)
                compute(pi, cj)

        pltpu.sync_copy(outbuf, out_hbm.at[pl.ds(p0 * c, ppw * c)])

    xt5 = xt.reshape(pairs, h, w, c)

    def tc_body(x_ref, o_ref):
        wm = jnp.logical_and(
            lax.broadcasted_iota(jnp.int32, (1, 1, w, 1), 2) >= 1,
            lax.broadcasted_iota(jnp.int32, (1, 1, w, 1), 2) <= w - 2,
        )
        xm = jnp.where(wm, x_ref[:, : h - 1], 0.0)
        o_ref[...] = jnp.sum(xm, axis=(1, 2)) * inv

    tcb = 32
    out_tc = pl.pallas_call(
        tc_body,
        grid=(p_tc // tcb,),
        in_specs=[
            pl.BlockSpec((tcb, h, w, c), lambda p: (p_sc // tcb + p, 0, 0, 0)),
        ],
        out_specs=pl.BlockSpec((tcb, c), lambda p: (p, 0)),
        out_shape=jax.ShapeDtypeStruct((p_tc, c), jnp.float32),
    )(xt5)

    out_sc = sc_kernel(xt)
    out = jnp.concatenate([out_sc.reshape(p_sc, c), out_tc], axis=0)
    return lax.transpose(out.reshape(b, t, c), (0, 2, 1))


def kernel(x):
    b, c, t, h, w = x.shape
    assert h == 16 and w == 16, "kernel specialized to 16x16 spatial grids"
    assert (b * t) % _NW == 0 and c % 16 == 0
    return _avg_pool(x, b, c, t, h, w)


# speedup vs baseline: 1.2022x; 1.0224x over previous
import functools

import jax
import jax.numpy as jnp
from jax import lax
from jax.experimental import pallas as pl
from jax.experimental.pallas import tpu as pltpu
from jax.experimental.pallas import tpu_sc as plsc

_NC = 2
_NS = 16
_NW = _NC * _NS


_SC_FRAC_NUM, _SC_FRAC_DEN = 4, 8


@functools.partial(jax.jit, static_argnames=("b", "c", "t", "h", "w"))
def _avg_pool(x, b, c, t, h, w):
    xt = lax.transpose(x, (0, 2, 3, 4, 1))
    pairs = b * t
    p_sc = pairs * _SC_FRAC_NUM // _SC_FRAC_DEN // _NW * _NW
    p_tc = pairs - p_sc
    n = p_sc * c
    ppw = p_sc // _NW
    nch = 2 * ppw
    cg = c // 16
    n_valid = (h - 1) * (h - 2)
    inv = 1.0 / float(n_valid)

    mesh = plsc.VectorSubcoreMesh(core_axis_name="c", subcore_axis_name="s")

    @functools.partial(
        pl.kernel,
        out_type=jax.ShapeDtypeStruct((n,), jnp.float32),
        mesh=mesh,
        scratch_types=[
            pltpu.VMEM((4, h, c), jnp.float32),
            pltpu.VMEM((4, h, c), jnp.float32),
            pltpu.VMEM((4, h, c), jnp.float32),
            pltpu.VMEM((3, h, c), jnp.float32),
            pltpu.VMEM((ppw * c,), jnp.float32),
            pltpu.SemaphoreType.DMA,
            pltpu.SemaphoreType.DMA,
            pltpu.SemaphoreType.DMA,
            pltpu.SemaphoreType.DMA,
        ],
    )
    def sc_kernel(xt_hbm, out_hbm, b0, b1, b2, b3, outbuf, s0, s1, s2, s3):
        wid = lax.axis_index("s") * _NC + lax.axis_index("c")
        p0 = wid * ppw
        bufs = [b0, b1, b2, b3]
        sems = [s0, s1, s2, s3]
        h0s = (0, 4, 8, 12)
        hss = (range(4), range(4), range(4), range(3))
        hns = (4, 4, 4, 3)

        def src(pi, cj):
            p = p0 + pi
            return xt_hbm.at[p // t, p % t, pl.ds(h0s[cj], hns[cj])]

        def start(pi, cj):
            @pl.when(pi < ppw)
            def _():
                pltpu.async_copy(src(pi, cj), bufs[cj], sems[cj])

        def wait(pi, cj):
            pltpu.make_async_copy(src(pi, cj), bufs[cj], sems[cj]).wait()

        def compute(pi, cj):
            buf = bufs[cj]

            @pl.loop(0, cg)
            def _cgrp(k):
                o = pl.ds(pi * c + k * 16, 16)
                vs = [
                    buf[hh, ww, pl.ds(k * 16, 16)]
                    for hh in hss[cj]
                    for ww in range(1, w - 1)
                ]
                if cj:
                    vs.append(outbuf[o])
                while len(vs) > 1:
                    nxt = [
                        vs[i] + vs[i + 1] for i in range(0, len(vs) - 1, 2)
                    ]
                    if len(vs) % 2:
                        nxt.append(vs[-1])
                    vs = nxt
                outbuf[o] = vs[0] * inv if cj == 3 else vs[0]

        for cj in range(3):
            start(0, cj)

        @pl.loop(0, ppw)
        def _plane(pi):
            for cj in range(4):
                nxt = cj + 3
                start(pi + nxt // 4, nxt % 4)
                wait(pi, cj)
                compute(pi, cj)

        pltpu.sync_copy(outbuf, out_hbm.at[pl.ds(p0 * c, ppw * c)])

    xt5 = xt.reshape(pairs, h, w, c)

    def tc_body(x_ref, o_ref):
        wm = jnp.logical_and(
            lax.broadcasted_iota(jnp.int32, (1, 1, w, 1), 2) >= 1,
            lax.broadcasted_iota(jnp.int32, (1, 1, w, 1), 2) <= w - 2,
        )
        xm = jnp.where(wm, x_ref[...], 0.0)
        o_ref[...] = jnp.sum(xm, axis=(1, 2)) * inv

    tcb = 32
    out_tc = pl.pallas_call(
        tc_body,
        grid=(p_tc // tcb,),
        in_specs=[
            pl.BlockSpec(
                (tcb, h - 1, w, c), lambda p: (p_sc // tcb + p, 0, 0, 0)
            ),
        ],
        out_specs=pl.BlockSpec((tcb, c), lambda p: (p, 0)),
        out_shape=jax.ShapeDtypeStruct((p_tc, c), jnp.float32),
    )(xt5)

    out_sc = sc_kernel(xt)
    out = jnp.concatenate([out_sc.reshape(p_sc, c), out_tc], axis=0)
    return lax.transpose(out.reshape(b, t, c), (0, 2, 1))


def kernel(x):
    b, c, t, h, w = x.shape
    assert h == 16 and w == 16, "kernel specialized to 16x16 spatial grids"
    assert (b * t) % _NW == 0 and c % 16 == 0
    return _avg_pool(x, b, c, t, h, w)


# speedup vs baseline: 1.2181x; 1.0132x over previous
import functools

import jax
import jax.numpy as jnp
from jax import lax
from jax.experimental import pallas as pl
from jax.experimental.pallas import tpu as pltpu
from jax.experimental.pallas import tpu_sc as plsc

_NC = 2
_NS = 16
_NW = _NC * _NS


_SC_FRAC_NUM, _SC_FRAC_DEN = 3, 8


@functools.partial(jax.jit, static_argnames=("b", "c", "t", "h", "w"))
def _avg_pool(x, b, c, t, h, w):
    xt = lax.transpose(x, (0, 2, 3, 4, 1))
    pairs = b * t
    p_sc = pairs * _SC_FRAC_NUM // _SC_FRAC_DEN // _NW * _NW
    p_tc = pairs - p_sc
    n = p_sc * c
    ppw = p_sc // _NW
    nch = 2 * ppw
    cg = c // 16
    n_valid = (h - 1) * (h - 2)
    inv = 1.0 / float(n_valid)

    mesh = plsc.VectorSubcoreMesh(core_axis_name="c", subcore_axis_name="s")

    @functools.partial(
        pl.kernel,
        out_type=jax.ShapeDtypeStruct((n,), jnp.float32),
        mesh=mesh,
        scratch_types=[
            pltpu.VMEM((4, h, c), jnp.float32),
            pltpu.VMEM((4, h, c), jnp.float32),
            pltpu.VMEM((4, h, c), jnp.float32),
            pltpu.VMEM((3, h, c), jnp.float32),
            pltpu.VMEM((ppw * c,), jnp.float32),
            pltpu.SemaphoreType.DMA,
            pltpu.SemaphoreType.DMA,
            pltpu.SemaphoreType.DMA,
            pltpu.SemaphoreType.DMA,
        ],
    )
    def sc_kernel(xt_hbm, out_hbm, b0, b1, b2, b3, outbuf, s0, s1, s2, s3):
        wid = lax.axis_index("s") * _NC + lax.axis_index("c")
        p0 = wid * ppw
        bufs = [b0, b1, b2, b3]
        sems = [s0, s1, s2, s3]
        h0s = (0, 4, 8, 12)
        hss = (range(4), range(4), range(4), range(3))
        hns = (4, 4, 4, 3)

        def src(pi, cj):
            p = p0 + pi
            return xt_hbm.at[p // t, p % t, pl.ds(h0s[cj], hns[cj])]

        def start(pi, cj):
            @pl.when(pi < ppw)
            def _():
                pltpu.async_copy(src(pi, cj), bufs[cj], sems[cj])

        def wait(pi, cj):
            pltpu.make_async_copy(src(pi, cj), bufs[cj], sems[cj]).wait()

        def compute(pi, cj):
            buf = bufs[cj]

            @pl.loop(0, cg)
            def _cgrp(k):
                o = pl.ds(pi * c + k * 16, 16)
                vs = [
                    buf[hh, ww, pl.ds(k * 16, 16)]
                    for hh in hss[cj]
                    for ww in range(1, w - 1)
                ]
                if cj:
                    vs.append(outbuf[o])
                while len(vs) > 1:
                    nxt = [
                        vs[i] + vs[i + 1] for i in range(0, len(vs) - 1, 2)
                    ]
                    if len(vs) % 2:
                        nxt.append(vs[-1])
                    vs = nxt
                outbuf[o] = vs[0] * inv if cj == 3 else vs[0]

        for cj in range(3):
            start(0, cj)

        @pl.loop(0, ppw)
        def _plane(pi):
            for cj in range(4):
                nxt = cj + 3
                start(pi + nxt // 4, nxt % 4)
                wait(pi, cj)
                compute(pi, cj)

        pltpu.sync_copy(outbuf, out_hbm.at[pl.ds(p0 * c, ppw * c)])

    xt5 = xt.reshape(pairs, h, w, c)

    def tc_body(x_ref, o_ref):
        wm = jnp.logical_and(
            lax.broadcasted_iota(jnp.int32, (1, 1, w, 1), 2) >= 1,
            lax.broadcasted_iota(jnp.int32, (1, 1, w, 1), 2) <= w - 2,
        )
        xm = jnp.where(wm, x_ref[...], 0.0)
        o_ref[...] = jnp.sum(xm, axis=(1, 2)) * inv

    tcb = 32
    out_tc = pl.pallas_call(
        tc_body,
        grid=(p_tc // tcb,),
        in_specs=[
            pl.BlockSpec(
                (tcb, h - 1, w, c), lambda p: (p_sc // tcb + p, 0, 0, 0)
            ),
        ],
        out_specs=pl.BlockSpec((tcb, c), lambda p: (p, 0)),
        out_shape=jax.ShapeDtypeStruct((p_tc, c), jnp.float32),
    )(xt5)

    out_sc = sc_kernel(xt)
    out = jnp.concatenate([out_sc.reshape(p_sc, c), out_tc], axis=0)
    return lax.transpose(out.reshape(b, t, c), (0, 2, 1))


def kernel(x):
    b, c, t, h, w = x.shape
    assert h == 16 and w == 16, "kernel specialized to 16x16 spatial grids"
    assert (b * t) % _NW == 0 and c % 16 == 0
    return _avg_pool(x, b, c, t, h, w)


# speedup vs baseline: 1.2414x; 1.0191x over previous
import functools

import jax
import jax.numpy as jnp
from jax import lax
from jax.experimental import pallas as pl
from jax.experimental.pallas import tpu as pltpu
from jax.experimental.pallas import tpu_sc as plsc

_NC = 2
_NS = 16
_NW = _NC * _NS


_SC_FRAC_NUM, _SC_FRAC_DEN = 2, 8


@functools.partial(jax.jit, static_argnames=("b", "c", "t", "h", "w"))
def _avg_pool(x, b, c, t, h, w):
    xt = lax.transpose(x, (0, 2, 3, 4, 1))
    pairs = b * t
    p_sc = pairs * _SC_FRAC_NUM // _SC_FRAC_DEN // _NW * _NW
    p_tc = pairs - p_sc
    n = p_sc * c
    ppw = p_sc // _NW
    nch = 2 * ppw
    cg = c // 16
    n_valid = (h - 1) * (h - 2)
    inv = 1.0 / float(n_valid)

    mesh = plsc.VectorSubcoreMesh(core_axis_name="c", subcore_axis_name="s")

    @functools.partial(
        pl.kernel,
        out_type=jax.ShapeDtypeStruct((n,), jnp.float32),
        mesh=mesh,
        scratch_types=[
            pltpu.VMEM((4, h, c), jnp.float32),
            pltpu.VMEM((4, h, c), jnp.float32),
            pltpu.VMEM((4, h, c), jnp.float32),
            pltpu.VMEM((3, h, c), jnp.float32),
            pltpu.VMEM((ppw * c,), jnp.float32),
            pltpu.SemaphoreType.DMA,
            pltpu.SemaphoreType.DMA,
            pltpu.SemaphoreType.DMA,
            pltpu.SemaphoreType.DMA,
        ],
    )
    def sc_kernel(xt_hbm, out_hbm, b0, b1, b2, b3, outbuf, s0, s1, s2, s3):
        wid = lax.axis_index("s") * _NC + lax.axis_index("c")
        p0 = wid * ppw
        bufs = [b0, b1, b2, b3]
        sems = [s0, s1, s2, s3]
        h0s = (0, 4, 8, 12)
        hss = (range(4), range(4), range(4), range(3))
        hns = (4, 4, 4, 3)

        def src(pi, cj):
            p = p0 + pi
            return xt_hbm.at[p // t, p % t, pl.ds(h0s[cj], hns[cj])]

        def start(pi, cj):
            @pl.when(pi < ppw)
            def _():
                pltpu.async_copy(src(pi, cj), bufs[cj], sems[cj])

        def wait(pi, cj):
            pltpu.make_async_copy(src(pi, cj), bufs[cj], sems[cj]).wait()

        def compute(pi, cj):
            buf = bufs[cj]

            @pl.loop(0, cg)
            def _cgrp(k):
                o = pl.ds(pi * c + k * 16, 16)
                vs = [
                    buf[hh, ww, pl.ds(k * 16, 16)]
                    for hh in hss[cj]
                    for ww in range(1, w - 1)
                ]
                if cj:
                    vs.append(outbuf[o])
                while len(vs) > 1:
                    nxt = [
                        vs[i] + vs[i + 1] for i in range(0, len(vs) - 1, 2)
                    ]
                    if len(vs) % 2:
                        nxt.append(vs[-1])
                    vs = nxt
                outbuf[o] = vs[0] * inv if cj == 3 else vs[0]

        for cj in range(3):
            start(0, cj)

        @pl.loop(0, ppw)
        def _plane(pi):
            for cj in range(4):
                nxt = cj + 3
                start(pi + nxt // 4, nxt % 4)
                wait(pi, cj)
                compute(pi, cj)

        pltpu.sync_copy(outbuf, out_hbm.at[pl.ds(p0 * c, ppw * c)])

    xt5 = xt.reshape(pairs, h, w, c)

    def tc_body(x_ref, o_ref):
        wm = jnp.logical_and(
            lax.broadcasted_iota(jnp.int32, (1, 1, w, 1), 2) >= 1,
            lax.broadcasted_iota(jnp.int32, (1, 1, w, 1), 2) <= w - 2,
        )
        xm = jnp.where(wm, x_ref[...], 0.0)
        o_ref[...] = jnp.sum(xm, axis=(1, 2)) * inv

    tcb = 32
    out_tc = pl.pallas_call(
        tc_body,
        grid=(p_tc // tcb,),
        in_specs=[
            pl.BlockSpec(
                (tcb, h - 1, w, c), lambda p: (p_sc // tcb + p, 0, 0, 0)
            ),
        ],
        out_specs=pl.BlockSpec((tcb, c), lambda p: (p, 0)),
        out_shape=jax.ShapeDtypeStruct((p_tc, c), jnp.float32),
    )(xt5)

    out_sc = sc_kernel(xt)
    out = jnp.concatenate([out_sc.reshape(p_sc, c), out_tc], axis=0)
    return lax.transpose(out.reshape(b, t, c), (0, 2, 1))


def kernel(x):
    b, c, t, h, w = x.shape
    assert h == 16 and w == 16, "kernel specialized to 16x16 spatial grids"
    assert (b * t) % _NW == 0 and c % 16 == 0
    return _avg_pool(x, b, c, t, h, w)
